# baseline (speedup 1.0000x reference)
import functools

import jax
import jax.numpy as jnp
from jax import lax
from jax.experimental import pallas as pl
from jax.experimental.pallas import tpu as pltpu
from jax.experimental.pallas import tpu_sc as plsc

N = 10000
E = 320000
D = 128
H = 512

NC = 2
NS = 16
NW = NC * NS

EDGES_PER_W = E // NW
BLK = 80
NSTEPS = EDGES_PER_W // BLK
NPAD = 10240
ROWS_PER_TILE = NPAD // NS
NBUF = 3


def _sc_scatter_kernel(edges_hbm, idx_hbm, out_hbm, ibuf, ebuf, shared,
                       sem_i, sem_z, sf0, sf1, sf2):
    sf = [sf0, sf1, sf2]
    cid = lax.axis_index("c")
    sid = lax.axis_index("s")
    wid = sid * NC + cid
    base = wid * EDGES_PER_W

    def _edge_slice(s):
        return edges_hbm.at[pl.ds(base + s * BLK, BLK), :]

    def _fetch_start(s, b):
        pltpu.async_copy(_edge_slice(s), ebuf.at[b], sf[b])

    def _fetch_wait(s, b):
        pltpu.make_async_copy(_edge_slice(s), ebuf.at[b], sf[b]).wait()

    idx_cp = pltpu.async_copy(idx_hbm.at[wid], ibuf, sem_i)

    zv = jnp.zeros((16,), jnp.float32)

    def _zero_row(i, _):
        def _zero_lane(k, _):
            ebuf[0, i, pl.ds(k * 16, 16)] = zv
            return 0
        return lax.fori_loop(0, D // 16, _zero_lane, 0)

    lax.fori_loop(0, BLK, _zero_row, 0)
    zcopies = []
    for c in range(ROWS_PER_TILE // BLK):
        zcopies.append(pltpu.async_copy(
            ebuf.at[0], shared.at[pl.ds(sid * ROWS_PER_TILE + c * BLK, BLK), :],
            sem_z))
    for cp in zcopies:
        cp.wait()
    plsc.subcore_barrier()

    for b in range(NBUF):
        _fetch_start(b, b)
    idx_cp.wait()

    def _do_step(s, b, prefetch):
        _fetch_wait(s, b)
        pltpu.sync_copy(ebuf.at[b], shared.at[ibuf.at[s]], add=True)
        if prefetch:
            _fetch_start(s + NBUF, b)

    def _outer(i, _):
        for b in range(NBUF):
            _do_step(i * NBUF + b, b, True)
        return 0

    lax.fori_loop(0, NSTEPS // NBUF - 1, _outer, 0)
    for s in range((NSTEPS // NBUF - 1) * NBUF, NSTEPS):
        _do_step(s, s % NBUF, s + NBUF < NSTEPS)

    plsc.subcore_barrier()
    pltpu.sync_copy(
        shared.at[pl.ds(sid * ROWS_PER_TILE, ROWS_PER_TILE), :],
        out_hbm.at[cid, pl.ds(sid * ROWS_PER_TILE, ROWS_PER_TILE), :],
    )


@jax.jit
def _sc_scatter(edge_features, src_indices):
    mesh = plsc.VectorSubcoreMesh(core_axis_name="c", subcore_axis_name="s")
    return pl.kernel(
        _sc_scatter_kernel,
        mesh=mesh,
        out_type=jax.ShapeDtypeStruct((NC, NPAD, D), jnp.float32),
        scratch_types=[
            pltpu.VMEM((NSTEPS, BLK), jnp.int32),
            pltpu.VMEM((NBUF, BLK, D), jnp.float32),
            pltpu.VMEM_SHARED((NPAD, D), jnp.float32),
        ] + [pltpu.SemaphoreType.DMA] * (NBUF + 2),
    )(edge_features, src_indices.reshape(NW, NSTEPS, BLK))


RB = 2000


def _pre_kernel(nf_ref, w1a_ref, b1_ref, out_ref):
    out_ref[...] = jnp.dot(nf_ref[...], w1a_ref[...],
                           preferred_element_type=jnp.float32) + b1_ref[...]


@jax.jit
def _pre(node_features, W1, b1):
    return pl.pallas_call(
        _pre_kernel,
        grid=(N // RB,),
        in_specs=[
            pl.BlockSpec((RB, D), lambda i: (i, 0)),
            pl.BlockSpec((D, H), lambda i: (0, 0)),
            pl.BlockSpec((1, H), lambda i: (0, 0)),
        ],
        out_specs=pl.BlockSpec((RB, H), lambda i: (i, 0)),
        out_shape=jax.ShapeDtypeStruct((N, H), jnp.float32),
    )(node_features, W1[:D], b1.reshape(1, H))


def _mlp_kernel(nf_ref, h1_ref, parts_ref, w1_ref, w2_ref, b2_ref, g_ref, bt_ref, out_ref):
    nf = nf_ref[...]
    agg = parts_ref[0] + parts_ref[1]
    h = h1_ref[...]
    h += jnp.dot(agg, w1_ref[...], preferred_element_type=jnp.float32)
    h = h * jax.nn.sigmoid(h)
    y = jnp.dot(h, w2_ref[...], preferred_element_type=jnp.float32) + b2_ref[...]
    mu = jnp.mean(y, axis=-1, keepdims=True)
    d = y - mu
    var = jnp.mean(d * d, axis=-1, keepdims=True)
    y = d * lax.rsqrt(var + 1e-5) * g_ref[...] + bt_ref[...]
    out_ref[...] = y + nf


@jax.jit
def _mlp(node_features, h1, parts, W1, W2, b2, gamma, beta):
    grid = (N // RB,)
    return pl.pallas_call(
        _mlp_kernel,
        grid=grid,
        in_specs=[
            pl.BlockSpec((RB, D), lambda i: (i, 0)),
            pl.BlockSpec((RB, H), lambda i: (i, 0)),
            pl.BlockSpec((NC, RB, D), lambda i: (0, i, 0)),
            pl.BlockSpec((D, H), lambda i: (0, 0)),
            pl.BlockSpec((H, D), lambda i: (0, 0)),
            pl.BlockSpec((1, D), lambda i: (0, 0)),
            pl.BlockSpec((1, D), lambda i: (0, 0)),
            pl.BlockSpec((1, D), lambda i: (0, 0)),
        ],
        out_specs=pl.BlockSpec((RB, D), lambda i: (i, 0)),
        out_shape=jax.ShapeDtypeStruct((N, D), jnp.float32),
    )(node_features, h1, parts, W1[D:], W2, b2.reshape(1, D),
      gamma.reshape(1, D), beta.reshape(1, D))


def kernel(node_features, edge_features, src_indices, W1, b1, W2, b2, gamma, beta):
    parts = _sc_scatter(edge_features, src_indices)
    h1 = _pre(node_features, W1, b1)
    return _mlp(node_features, h1, parts, W1, W2, b2, gamma, beta)


# speedup vs baseline: 1.0694x; 1.0694x over previous
import functools

import jax
import jax.numpy as jnp
from jax import lax
from jax.experimental import pallas as pl
from jax.experimental.pallas import tpu as pltpu
from jax.experimental.pallas import tpu_sc as plsc

N = 10000
E = 320000
D = 128
H = 512

NC = 2
NS = 16
NW = NC * NS

EDGES_PER_W = E // NW
BLK = 80
NSTEPS = EDGES_PER_W // BLK
NPAD = 10240
ROWS_PER_TILE = NPAD // NS
NBUF = 3


def _sc_scatter_kernel(edges_hbm, idx_hbm, out_hbm, ibuf, ebuf, shared,
                       sem_i, sem_z, sf0, sf1, sf2):
    sf = [sf0, sf1, sf2]
    cid = lax.axis_index("c")
    sid = lax.axis_index("s")
    wid = sid * NC + cid
    base = wid * EDGES_PER_W

    def _edge_slice(s):
        return edges_hbm.at[pl.ds(base + s * BLK, BLK), :]

    def _fetch_start(s, b):
        pltpu.async_copy(_edge_slice(s), ebuf.at[b], sf[b])

    def _fetch_wait(s, b):
        pltpu.make_async_copy(_edge_slice(s), ebuf.at[b], sf[b]).wait()

    idx_cp = pltpu.async_copy(idx_hbm.at[wid], ibuf, sem_i)

    zv = jnp.zeros((16,), jnp.float32)

    def _zero_row(i, _):
        def _zero_lane(k, _):
            ebuf[0, i, pl.ds(k * 16, 16)] = zv
            return 0
        return lax.fori_loop(0, D // 16, _zero_lane, 0)

    lax.fori_loop(0, BLK, _zero_row, 0)
    zcopies = []
    for c in range(ROWS_PER_TILE // BLK):
        zcopies.append(pltpu.async_copy(
            ebuf.at[0], shared.at[pl.ds(sid * ROWS_PER_TILE + c * BLK, BLK), :],
            sem_z))
    for cp in zcopies:
        cp.wait()
    plsc.subcore_barrier()

    for b in range(NBUF):
        _fetch_start(b, b)
    idx_cp.wait()

    def _do_step(s, b, prefetch):
        _fetch_wait(s, b)
        pltpu.sync_copy(ebuf.at[b], shared.at[ibuf.at[s]], add=True)
        if prefetch:
            _fetch_start(s + NBUF, b)

    def _outer(i, _):
        for b in range(NBUF):
            _do_step(i * NBUF + b, b, True)
        return 0

    lax.fori_loop(0, NSTEPS // NBUF - 1, _outer, 0)
    for s in range((NSTEPS // NBUF - 1) * NBUF, NSTEPS):
        _do_step(s, s % NBUF, s + NBUF < NSTEPS)

    plsc.subcore_barrier()
    pltpu.sync_copy(
        shared.at[pl.ds(sid * ROWS_PER_TILE, ROWS_PER_TILE), :],
        out_hbm.at[cid, pl.ds(sid * ROWS_PER_TILE, ROWS_PER_TILE), :],
    )


@jax.jit
def _sc_scatter(edge_features, src_indices):
    mesh = plsc.VectorSubcoreMesh(core_axis_name="c", subcore_axis_name="s")
    return pl.kernel(
        _sc_scatter_kernel,
        mesh=mesh,
        out_type=jax.ShapeDtypeStruct((NC, NPAD, D), jnp.float32),
        scratch_types=[
            pltpu.VMEM((NSTEPS, BLK), jnp.int32),
            pltpu.VMEM((NBUF, BLK, D), jnp.float32),
            pltpu.VMEM_SHARED((NPAD, D), jnp.float32),
        ] + [pltpu.SemaphoreType.DMA] * (NBUF + 2),
    )(edge_features, src_indices.reshape(NW, NSTEPS, BLK))


RB = 2000


def _mlp_kernel(nf_ref, parts_ref, w1_ref, b1_ref, w2_ref, b2_ref, g_ref, bt_ref, out_ref):
    nf = nf_ref[...]
    agg = parts_ref[0] + parts_ref[1]
    w1 = w1_ref[...]
    h = jnp.dot(nf, w1[:D], preferred_element_type=jnp.float32)
    h += jnp.dot(agg, w1[D:], preferred_element_type=jnp.float32)
    h += b1_ref[...]
    h = h * jax.nn.sigmoid(h)
    y = jnp.dot(h, w2_ref[...], preferred_element_type=jnp.float32) + b2_ref[...]
    mu = jnp.mean(y, axis=-1, keepdims=True)
    d = y - mu
    var = jnp.mean(d * d, axis=-1, keepdims=True)
    y = d * lax.rsqrt(var + 1e-5) * g_ref[...] + bt_ref[...]
    out_ref[...] = y + nf


@jax.jit
def _mlp(node_features, parts, W1, b1, W2, b2, gamma, beta):
    grid = (N // RB,)
    return pl.pallas_call(
        _mlp_kernel,
        grid=grid,
        in_specs=[
            pl.BlockSpec((RB, D), lambda i: (i, 0)),
            pl.BlockSpec((NC, RB, D), lambda i: (0, i, 0)),
            pl.BlockSpec((2 * D, H), lambda i: (0, 0)),
            pl.BlockSpec((1, H), lambda i: (0, 0)),
            pl.BlockSpec((H, D), lambda i: (0, 0)),
            pl.BlockSpec((1, D), lambda i: (0, 0)),
            pl.BlockSpec((1, D), lambda i: (0, 0)),
            pl.BlockSpec((1, D), lambda i: (0, 0)),
        ],
        out_specs=pl.BlockSpec((RB, D), lambda i: (i, 0)),
        out_shape=jax.ShapeDtypeStruct((N, D), jnp.float32),
    )(node_features, parts, W1, b1.reshape(1, H), W2, b2.reshape(1, D),
      gamma.reshape(1, D), beta.reshape(1, D))


def kernel(node_features, edge_features, src_indices, W1, b1, W2, b2, gamma, beta):
    parts = _sc_scatter(edge_features, src_indices)
    return _mlp(node_features, parts, W1, b1, W2, b2, gamma, beta)
